# Initial kernel scaffold; baseline (speedup 1.0000x reference)
import jax
import jax.numpy as jnp
from jax.experimental import pallas as pl


def kernel(feature, edge_index, W_gcn, W_enh):
    raise NotImplementedError("write your pallas kernel here")


# speedup vs baseline: 14.1470x; 14.1470x over previous
import functools

import jax
import jax.numpy as jnp
from jax import lax
from jax.experimental import pallas as pl
from jax.experimental.pallas import tpu as pltpu
from jax.experimental.pallas import tpu_sc as plsc

N = 10000
E = 320000
D = 128
NC = 2
NS = 16
NW = NC * NS
NP = 10240
EPW = E // NW
CH = 80
NCHUNK = EPW // CH
RPT = NP // NS
ZCH = 128
NZ = RPT // ZCH
SW = 16
RB = 1000
NRB = N // RB

_mesh = plsc.VectorSubcoreMesh(
    core_axis_name="c", subcore_axis_name="s", num_cores=NC, num_subcores=NS)


def _zero_fill(buf, rows, width):
  def body(i, _):
    for j in range(width // 16):
      buf[i, pl.ds(j * 16, 16)] = jnp.zeros((16,), jnp.float32)
    return 0
  lax.fori_loop(0, rows, body, 0, unroll=False)


@functools.partial(
    pl.kernel,
    out_type=jax.ShapeDtypeStruct((NC, NP, SW), jnp.float32),
    mesh=_mesh,
    scratch_types=[
        pltpu.VMEM_SHARED((NP, SW), jnp.float32),
        pltpu.VMEM((CH,), jnp.int32),
        pltpu.VMEM((CH, SW), jnp.float32),
        pltpu.VMEM((ZCH, SW), jnp.float32),
    ],
    compiler_params=pltpu.CompilerParams(use_tc_tiling_on_sc=False),
)
def _sc_degree(dst_hbm, deg_out, shared_deg, idx_v, ones_v, zrow_v):
  cid = lax.axis_index("c")
  sid = lax.axis_index("s")
  wid = sid * NC + cid

  def fill_ones(i, _):
    ones_v[i, :] = jnp.full((SW,), 1.0, jnp.float32)
    return 0
  lax.fori_loop(0, CH, fill_ones, 0, unroll=False)
  _zero_fill(zrow_v, ZCH, SW)

  for z in range(NZ):
    r0 = pl.multiple_of(sid * RPT + z * ZCH, ZCH)
    pltpu.sync_copy(zrow_v, shared_deg.at[pl.ds(r0, ZCH)])
  plsc.subcore_barrier()

  def chunk(i, _):
    base = pl.multiple_of(wid * EPW + i * CH, 8)
    pltpu.sync_copy(dst_hbm.at[pl.ds(base, CH)], idx_v)
    pltpu.sync_copy(ones_v, shared_deg.at[idx_v], add=True)
    return 0
  lax.fori_loop(0, NCHUNK, chunk, 0, unroll=False)
  plsc.subcore_barrier()

  for z in range(NZ):
    r0 = pl.multiple_of(sid * RPT + z * ZCH, ZCH)
    pltpu.sync_copy(shared_deg.at[pl.ds(r0, ZCH)], zrow_v)
    pltpu.sync_copy(zrow_v, deg_out.at[cid, pl.ds(r0, ZCH)])


@functools.partial(
    pl.kernel,
    out_type=(
        jax.ShapeDtypeStruct((NC, NP, D), jnp.float32),
        jax.ShapeDtypeStruct((NC, NP, SW), jnp.float32),
    ),
    mesh=_mesh,
    scratch_types=[
        pltpu.VMEM_SHARED((NP, D), jnp.float32),
        pltpu.VMEM_SHARED((NP, SW), jnp.float32),
        pltpu.VMEM((CH,), jnp.int32),
        pltpu.VMEM((CH,), jnp.int32),
        pltpu.VMEM((CH, D), jnp.float32),
        pltpu.VMEM((CH, SW), jnp.float32),
        pltpu.VMEM((ZCH, D), jnp.float32),
        pltpu.VMEM((ZCH, SW), jnp.float32),
        pltpu.SemaphoreType.DMA,
    ],
    compiler_params=pltpu.CompilerParams(use_tc_tiling_on_sc=False),
)
def _sc_edges(src_hbm, dst_hbm, h_hbm, isq_hbm, g_out, s1_out,
              shared_g, shared_s1, idxs_v, idxd_v, rows_v, sca_v,
              zbuf_v, zsca_v, sem):
  cid = lax.axis_index("c")
  sid = lax.axis_index("s")
  wid = sid * NC + cid

  _zero_fill(zbuf_v, ZCH, D)
  _zero_fill(zsca_v, ZCH, SW)
  for z in range(NZ):
    r0 = pl.multiple_of(sid * RPT + z * ZCH, ZCH)
    pltpu.sync_copy(zbuf_v, shared_g.at[pl.ds(r0, ZCH)])
    pltpu.sync_copy(zsca_v, shared_s1.at[pl.ds(r0, ZCH)])
  plsc.subcore_barrier()

  def chunk(i, _):
    base = pl.multiple_of(wid * EPW + i * CH, 8)
    pltpu.sync_copy(src_hbm.at[pl.ds(base, CH)], idxs_v)
    pltpu.sync_copy(dst_hbm.at[pl.ds(base, CH)], idxd_v)
    pltpu.async_copy(h_hbm.at[idxs_v], rows_v, sem).wait()
    pltpu.async_copy(isq_hbm.at[idxs_v], sca_v, sem).wait()
    pltpu.sync_copy(rows_v, shared_g.at[idxd_v], add=True)
    pltpu.sync_copy(sca_v, shared_s1.at[idxd_v], add=True)
    return 0
  lax.fori_loop(0, NCHUNK, chunk, 0, unroll=False)
  plsc.subcore_barrier()

  for z in range(NZ):
    r0 = pl.multiple_of(sid * RPT + z * ZCH, ZCH)
    pltpu.sync_copy(shared_g.at[pl.ds(r0, ZCH)], zbuf_v)
    pltpu.sync_copy(zbuf_v, g_out.at[cid, pl.ds(r0, ZCH)])
    pltpu.sync_copy(shared_s1.at[pl.ds(r0, ZCH)], zsca_v)
    pltpu.sync_copy(zsca_v, s1_out.at[cid, pl.ds(r0, ZCH)])


def _tc_prep_body(degA_ref, degB_ref, f_ref, h_ref, isq16_ref):
  deg = degA_ref[:, 0:1] + degB_ref[:, 0:1]
  isq = lax.rsqrt(jnp.maximum(deg, 1.0))
  h_ref[...] = f_ref[...] * isq
  isq16_ref[...] = jnp.broadcast_to(isq, (RB, SW))


def _tc_prep(degA, degB, feature):
  return pl.pallas_call(
      _tc_prep_body,
      grid=(NRB,),
      in_specs=[
          pl.BlockSpec((RB, SW), lambda i: (i, 0)),
          pl.BlockSpec((RB, SW), lambda i: (i, 0)),
          pl.BlockSpec((RB, D), lambda i: (i, 0)),
      ],
      out_specs=[
          pl.BlockSpec((RB, D), lambda i: (i, 0)),
          pl.BlockSpec((RB, SW), lambda i: (i, 0)),
      ],
      out_shape=[
          jax.ShapeDtypeStruct((N, D), jnp.float32),
          jax.ShapeDtypeStruct((N, SW), jnp.float32),
      ],
  )(degA, degB, feature)


def _tc_final_body(f_ref, g0_ref, g1_ref, s1a_ref, s1b_ref, isq16_ref,
                   wg_ref, we_ref, o_ref):
  isq = isq16_ref[:, 0:1]
  s1 = s1a_ref[:, 0:1] + s1b_ref[:, 0:1]
  f = f_ref[...]
  a = f * (1.0 + isq * s1)
  b = f * (g0_ref[...] + g1_ref[...]) * isq
  dn = (((1,), (1,)), ((), ()))
  r = lax.dot_general(a, wg_ref[...], dn,
                      preferred_element_type=jnp.float32,
                      precision=lax.Precision.HIGHEST)
  r = r + lax.dot_general(b, we_ref[...], dn,
                          preferred_element_type=jnp.float32,
                          precision=lax.Precision.HIGHEST)
  r = jnp.where(r >= 0, r, 0.2 * r)
  nrm = jnp.sqrt(jnp.sum(r * r, axis=1, keepdims=True))
  o_ref[...] = r / jnp.maximum(nrm, 1e-12)


def _tc_final(feature, g0, g1, s1a, s1b, isq16, W_gcn, W_enh):
  return pl.pallas_call(
      _tc_final_body,
      grid=(NRB,),
      in_specs=[
          pl.BlockSpec((RB, D), lambda i: (i, 0)),
          pl.BlockSpec((RB, D), lambda i: (i, 0)),
          pl.BlockSpec((RB, D), lambda i: (i, 0)),
          pl.BlockSpec((RB, SW), lambda i: (i, 0)),
          pl.BlockSpec((RB, SW), lambda i: (i, 0)),
          pl.BlockSpec((RB, SW), lambda i: (i, 0)),
          pl.BlockSpec((D, D), lambda i: (0, 0)),
          pl.BlockSpec((D, D), lambda i: (0, 0)),
      ],
      out_specs=pl.BlockSpec((RB, D), lambda i: (i, 0)),
      out_shape=jax.ShapeDtypeStruct((N, D), jnp.float32),
  )(feature, g0, g1, s1a, s1b, isq16, W_gcn, W_enh)


def kernel(feature, edge_index, W_gcn, W_enh):
  ei = edge_index.astype(jnp.int32)
  src = ei[0]
  dst = ei[1]
  deg_p = _sc_degree(dst)
  h, isq16 = _tc_prep(deg_p[0, :N], deg_p[1, :N], feature)
  g_p, s1_p = _sc_edges(src, dst, h, isq16)
  return _tc_final(feature, g_p[0, :N], g_p[1, :N], s1_p[0, :N], s1_p[1, :N],
                   isq16, W_gcn, W_enh)


# speedup vs baseline: 28.2353x; 1.9958x over previous
import functools

import jax
import jax.numpy as jnp
from jax import lax
from jax.experimental import pallas as pl
from jax.experimental.pallas import tpu as pltpu
from jax.experimental.pallas import tpu_sc as plsc

N = 10000
E = 320000
D = 128
DE = 144
NC = 2
NS = 16
NW = NC * NS
NP = 10240
CH = 128
NCHG = E // CH
NKB = NCHG // NW
NKR = NCHG % NW
RPT = NP // NS
ZCH = 128
NZ = RPT // ZCH
SW = 16
RB = 1000
NRB = N // RB

_mesh = plsc.VectorSubcoreMesh(
    core_axis_name="c", subcore_axis_name="s", num_cores=NC, num_subcores=NS)


def _zero_fill(buf, rows, width):
  def body(i, _):
    for j in range(width // 16):
      buf[i, pl.ds(j * 16, 16)] = jnp.zeros((16,), jnp.float32)
    return 0
  lax.fori_loop(0, rows, body, 0, unroll=False)


def _chunk_base(wid, k):
  return pl.multiple_of((wid + NW * k) * CH, 8)


@functools.partial(
    pl.kernel,
    out_type=jax.ShapeDtypeStruct((NC, NP, SW), jnp.float32),
    mesh=_mesh,
    scratch_types=[
        pltpu.VMEM_SHARED((NP, SW), jnp.float32),
        pltpu.VMEM((CH,), jnp.int32),
        pltpu.VMEM((CH,), jnp.int32),
        pltpu.VMEM((CH, SW), jnp.float32),
        pltpu.VMEM((ZCH, SW), jnp.float32),
        pltpu.SemaphoreType.DMA,
        pltpu.SemaphoreType.DMA,
    ],
    compiler_params=pltpu.CompilerParams(use_tc_tiling_on_sc=False),
)
def _sc_degree(dst_hbm, deg_out, shared_deg, idx0, idx1, ones_v, zrow_v,
               isem0, isem1):
  cid = lax.axis_index("c")
  sid = lax.axis_index("s")
  wid = sid * NC + cid
  nk = NKB + jnp.where(wid < NKR, 1, 0)

  def fill_ones(i, _):
    ones_v[i, :] = jnp.full((SW,), 1.0, jnp.float32)
    return 0
  lax.fori_loop(0, CH, fill_ones, 0, unroll=False)
  _zero_fill(zrow_v, ZCH, SW)

  for z in range(NZ):
    r0 = pl.multiple_of(sid * RPT + z * ZCH, ZCH)
    pltpu.sync_copy(zrow_v, shared_deg.at[pl.ds(r0, ZCH)])
  plsc.subcore_barrier()

  bufs = ((idx0, isem0), (idx1, isem1))

  def fire_idx(b, k):
    idx, isem = bufs[b]
    pltpu.async_copy(dst_hbm.at[pl.ds(_chunk_base(wid, k), CH)], idx, isem)

  def wait_idx(b):
    idx, isem = bufs[b]
    pltpu.make_async_copy(dst_hbm.at[pl.ds(0, CH)], idx, isem).wait()

  fire_idx(0, 0)

  @pl.when(nk > 1)
  def _():
    fire_idx(1, 1)

  def body(k, _):
    def step(a, b):
      idx_a, _ = bufs[a]
      wait_idx(a)
      pltpu.sync_copy(ones_v, shared_deg.at[idx_a], add=True)

      @pl.when(k + 2 < nk)
      def _():
        fire_idx(a, k + 2)

    @pl.when(k % 2 == 0)
    def _():
      step(0, 1)

    @pl.when(k % 2 == 1)
    def _():
      step(1, 0)
    return 0
  lax.fori_loop(0, nk, body, 0, unroll=False)
  plsc.subcore_barrier()

  for z in range(NZ):
    r0 = pl.multiple_of(sid * RPT + z * ZCH, ZCH)
    pltpu.sync_copy(shared_deg.at[pl.ds(r0, ZCH)], zrow_v)
    pltpu.sync_copy(zrow_v, deg_out.at[cid, pl.ds(r0, ZCH)])


@functools.partial(
    pl.kernel,
    out_type=jax.ShapeDtypeStruct((NC, NP, DE), jnp.float32),
    mesh=_mesh,
    scratch_types=[
        pltpu.VMEM_SHARED((NP, DE), jnp.float32),
        pltpu.VMEM((CH,), jnp.int32),
        pltpu.VMEM((CH,), jnp.int32),
        pltpu.VMEM((CH,), jnp.int32),
        pltpu.VMEM((CH,), jnp.int32),
        pltpu.VMEM((CH, DE), jnp.float32),
        pltpu.VMEM((CH, DE), jnp.float32),
        pltpu.SemaphoreType.DMA,
        pltpu.SemaphoreType.DMA,
        pltpu.SemaphoreType.DMA,
        pltpu.SemaphoreType.DMA,
    ],
    compiler_params=pltpu.CompilerParams(use_tc_tiling_on_sc=False),
)
def _sc_edges(src_hbm, dst_hbm, hext_hbm, gext_out, shared_g,
              idxs0, idxd0, idxs1, idxd1, rows0, rows1,
              isem0, isem1, gsem0, gsem1):
  zbuf_v = rows0
  cid = lax.axis_index("c")
  sid = lax.axis_index("s")
  wid = sid * NC + cid
  nk = NKB + jnp.where(wid < NKR, 1, 0)

  _zero_fill(zbuf_v, ZCH, DE)
  for z in range(NZ):
    r0 = pl.multiple_of(sid * RPT + z * ZCH, ZCH)
    pltpu.sync_copy(zbuf_v, shared_g.at[pl.ds(r0, ZCH)])
  plsc.subcore_barrier()

  bufs = ((idxs0, idxd0, rows0, isem0, gsem0),
          (idxs1, idxd1, rows1, isem1, gsem1))

  def fire_idx(b, k):
    idxs, idxd, _, isem, _ = bufs[b]
    base = _chunk_base(wid, k)
    pltpu.async_copy(src_hbm.at[pl.ds(base, CH)], idxs, isem)
    pltpu.async_copy(dst_hbm.at[pl.ds(base, CH)], idxd, isem)

  def wait_idx(b):
    idxs, idxd, _, isem, _ = bufs[b]
    pltpu.make_async_copy(src_hbm.at[pl.ds(0, CH)], idxs, isem).wait()
    pltpu.make_async_copy(dst_hbm.at[pl.ds(0, CH)], idxd, isem).wait()

  def fire_gather(b):
    idxs, _, rows, _, gsem = bufs[b]
    pltpu.async_copy(hext_hbm.at[idxs], rows, gsem)

  def wait_gather(b):
    idxs, _, rows, _, gsem = bufs[b]
    pltpu.make_async_copy(hext_hbm.at[idxs], rows, gsem).wait()

  fire_idx(0, 0)
  wait_idx(0)
  fire_gather(0)

  @pl.when(nk > 1)
  def _():
    fire_idx(1, 1)

  def body(k, _):
    def step(a, b):
      _, idxd_a, rows_a, _, _ = bufs[a]
      wait_gather(a)

      @pl.when(k + 1 < nk)
      def _():
        wait_idx(b)
        fire_gather(b)

      pltpu.sync_copy(rows_a, shared_g.at[idxd_a], add=True)

      @pl.when(k + 2 < nk)
      def _():
        fire_idx(a, k + 2)

    @pl.when(k % 2 == 0)
    def _():
      step(0, 1)

    @pl.when(k % 2 == 1)
    def _():
      step(1, 0)
    return 0
  lax.fori_loop(0, nk, body, 0, unroll=False)
  plsc.subcore_barrier()

  for z in range(NZ):
    r0 = pl.multiple_of(sid * RPT + z * ZCH, ZCH)
    pltpu.sync_copy(shared_g.at[pl.ds(r0, ZCH)], zbuf_v)
    pltpu.sync_copy(zbuf_v, gext_out.at[cid, pl.ds(r0, ZCH)])


def _tc_prep_body(degA_ref, degB_ref, f_ref, h_ref):
  deg = degA_ref[:, 0:1] + degB_ref[:, 0:1]
  isq = lax.rsqrt(jnp.maximum(deg, 1.0))
  h = jnp.concatenate(
      [f_ref[...] * isq, isq, jnp.zeros((RB, DE - D - 1), jnp.float32)],
      axis=1)
  h_ref[...] = h


def _tc_prep(degA, degB, feature):
  return pl.pallas_call(
      _tc_prep_body,
      grid=(NRB,),
      in_specs=[
          pl.BlockSpec((RB, SW), lambda i: (i, 0)),
          pl.BlockSpec((RB, SW), lambda i: (i, 0)),
          pl.BlockSpec((RB, D), lambda i: (i, 0)),
      ],
      out_specs=pl.BlockSpec((RB, DE), lambda i: (i, 0)),
      out_shape=jax.ShapeDtypeStruct((N, DE), jnp.float32),
  )(degA, degB, feature)


def _tc_final_body(f_ref, h_ref, g0_ref, g1_ref, wg_ref, we_ref, o_ref):
  gext = g0_ref[...] + g1_ref[...]
  hf = h_ref[:, 0:D]
  s1 = gext[:, D:D + 1]
  g = gext[:, 0:D]
  f = f_ref[...]
  a = f + hf * s1
  b = hf * g
  dn = (((1,), (1,)), ((), ()))
  r = lax.dot_general(a, wg_ref[...], dn,
                      preferred_element_type=jnp.float32,
                      precision=lax.Precision.HIGHEST)
  r = r + lax.dot_general(b, we_ref[...], dn,
                          preferred_element_type=jnp.float32,
                          precision=lax.Precision.HIGHEST)
  r = jnp.where(r >= 0, r, 0.2 * r)
  nrm = jnp.sqrt(jnp.sum(r * r, axis=1, keepdims=True))
  o_ref[...] = r / jnp.maximum(nrm, 1e-12)


def _tc_final(feature, hext, g0, g1, W_gcn, W_enh):
  return pl.pallas_call(
      _tc_final_body,
      grid=(NRB,),
      in_specs=[
          pl.BlockSpec((RB, D), lambda i: (i, 0)),
          pl.BlockSpec((RB, DE), lambda i: (i, 0)),
          pl.BlockSpec((RB, DE), lambda i: (i, 0)),
          pl.BlockSpec((RB, DE), lambda i: (i, 0)),
          pl.BlockSpec((D, D), lambda i: (0, 0)),
          pl.BlockSpec((D, D), lambda i: (0, 0)),
      ],
      out_specs=pl.BlockSpec((RB, D), lambda i: (i, 0)),
      out_shape=jax.ShapeDtypeStruct((N, D), jnp.float32),
  )(feature, hext, g0, g1, W_gcn, W_enh)


def kernel(feature, edge_index, W_gcn, W_enh):
  ei = edge_index.astype(jnp.int32)
  src = ei[0]
  dst = ei[1]
  deg_p = _sc_degree(dst)
  hext = _tc_prep(deg_p[0, :N], deg_p[1, :N], feature)
  g_p = _sc_edges(src, dst, hext)
  return _tc_final(feature, hext, g_p[0, :N], g_p[1, :N], W_gcn, W_enh)


# speedup vs baseline: 31.7145x; 1.1232x over previous
import functools

import jax
import jax.numpy as jnp
from jax import lax
from jax.experimental import pallas as pl
from jax.experimental.pallas import tpu as pltpu
from jax.experimental.pallas import tpu_sc as plsc

N = 10000
E = 320000
D = 128
DE = 144
NC = 2
NS = 16
NW = NC * NS
NP = 10240
CH = 128
NCHG = E // CH
NKB = NCHG // NW
NKR = NCHG % NW
RPT = NP // NS
ZCH = 128
NZ = RPT // ZCH
SW = 16
RB = 1000
NRB = N // RB

_mesh = plsc.VectorSubcoreMesh(
    core_axis_name="c", subcore_axis_name="s", num_cores=NC, num_subcores=NS)


def _zero_fill(buf, rows, width):
  def body(i, _):
    for j in range(width // 16):
      buf[i, pl.ds(j * 16, 16)] = jnp.zeros((16,), jnp.float32)
    return 0
  lax.fori_loop(0, rows, body, 0, unroll=False)


def _chunk_base(wid, k):
  return pl.multiple_of((wid + NW * k) * CH, 8)


@functools.partial(
    pl.kernel,
    out_type=jax.ShapeDtypeStruct((NC, NP, SW), jnp.float32),
    mesh=_mesh,
    scratch_types=[
        pltpu.VMEM_SHARED((NP, SW), jnp.float32),
        pltpu.VMEM((CH,), jnp.int32),
        pltpu.VMEM((CH,), jnp.int32),
        pltpu.VMEM((CH, SW), jnp.float32),
        pltpu.VMEM((ZCH, SW), jnp.float32),
        pltpu.SemaphoreType.DMA,
        pltpu.SemaphoreType.DMA,
    ],
    compiler_params=pltpu.CompilerParams(use_tc_tiling_on_sc=False),
)
def _sc_degree(ei_hbm, deg_out, shared_deg, idx0, idx1, ones_v, zrow_v,
               isem0, isem1):
  cid = lax.axis_index("c")
  sid = lax.axis_index("s")
  wid = sid * NC + cid
  nk = NKB + jnp.where(wid < NKR, 1, 0)

  def fill_ones(i, _):
    ones_v[i, :] = jnp.full((SW,), 1.0, jnp.float32)
    return 0
  lax.fori_loop(0, CH, fill_ones, 0, unroll=False)
  _zero_fill(zrow_v, ZCH, SW)

  for z in range(NZ):
    r0 = pl.multiple_of(sid * RPT + z * ZCH, ZCH)
    pltpu.sync_copy(zrow_v, shared_deg.at[pl.ds(r0, ZCH)])
  plsc.subcore_barrier()

  bufs = ((idx0, isem0), (idx1, isem1))

  def fire_idx(b, k):
    idx, isem = bufs[b]
    pltpu.async_copy(ei_hbm.at[1, pl.ds(_chunk_base(wid, k), CH)], idx, isem)

  def wait_idx(b):
    idx, isem = bufs[b]
    pltpu.make_async_copy(ei_hbm.at[1, pl.ds(0, CH)], idx, isem).wait()

  fire_idx(0, 0)

  @pl.when(nk > 1)
  def _():
    fire_idx(1, 1)

  def body(k, _):
    def step(a, b):
      idx_a, _ = bufs[a]
      wait_idx(a)
      pltpu.sync_copy(ones_v, shared_deg.at[idx_a], add=True)

      @pl.when(k + 2 < nk)
      def _():
        fire_idx(a, k + 2)

    @pl.when(k % 2 == 0)
    def _():
      step(0, 1)

    @pl.when(k % 2 == 1)
    def _():
      step(1, 0)
    return 0
  lax.fori_loop(0, nk, body, 0, unroll=False)
  plsc.subcore_barrier()

  for z in range(NZ):
    r0 = pl.multiple_of(sid * RPT + z * ZCH, ZCH)
    pltpu.sync_copy(shared_deg.at[pl.ds(r0, ZCH)], zrow_v)
    pltpu.sync_copy(zrow_v, deg_out.at[cid, pl.ds(r0, ZCH)])


@functools.partial(
    pl.kernel,
    out_type=jax.ShapeDtypeStruct((NC, NP, DE), jnp.float32),
    mesh=_mesh,
    scratch_types=[
        pltpu.VMEM_SHARED((NP, DE), jnp.float32),
        pltpu.VMEM((CH,), jnp.int32),
        pltpu.VMEM((CH,), jnp.int32),
        pltpu.VMEM((CH,), jnp.int32),
        pltpu.VMEM((CH,), jnp.int32),
        pltpu.VMEM((CH, DE), jnp.float32),
        pltpu.VMEM((CH, DE), jnp.float32),
        pltpu.SemaphoreType.DMA,
        pltpu.SemaphoreType.DMA,
        pltpu.SemaphoreType.DMA,
        pltpu.SemaphoreType.DMA,
    ],
    compiler_params=pltpu.CompilerParams(use_tc_tiling_on_sc=False),
)
def _sc_edges(ei_hbm, hext_hbm, gext_out, shared_g,
              idxs0, idxd0, idxs1, idxd1, rows0, rows1,
              isem0, isem1, gsem0, gsem1):
  zbuf_v = rows0
  cid = lax.axis_index("c")
  sid = lax.axis_index("s")
  wid = sid * NC + cid
  nk = NKB + jnp.where(wid < NKR, 1, 0)

  _zero_fill(zbuf_v, ZCH, DE)
  for z in range(NZ):
    r0 = pl.multiple_of(sid * RPT + z * ZCH, ZCH)
    pltpu.sync_copy(zbuf_v, shared_g.at[pl.ds(r0, ZCH)])
  plsc.subcore_barrier()

  bufs = ((idxs0, idxd0, rows0, isem0, gsem0),
          (idxs1, idxd1, rows1, isem1, gsem1))

  def fire_idx(b, k):
    idxs, idxd, _, isem, _ = bufs[b]
    base = _chunk_base(wid, k)
    pltpu.async_copy(ei_hbm.at[0, pl.ds(base, CH)], idxs, isem)
    pltpu.async_copy(ei_hbm.at[1, pl.ds(base, CH)], idxd, isem)

  def wait_idx(b):
    idxs, idxd, _, isem, _ = bufs[b]
    pltpu.make_async_copy(ei_hbm.at[0, pl.ds(0, CH)], idxs, isem).wait()
    pltpu.make_async_copy(ei_hbm.at[1, pl.ds(0, CH)], idxd, isem).wait()

  def fire_gather(b):
    idxs, _, rows, _, gsem = bufs[b]
    pltpu.async_copy(hext_hbm.at[idxs], rows, gsem)

  def wait_gather(b):
    idxs, _, rows, _, gsem = bufs[b]
    pltpu.make_async_copy(hext_hbm.at[idxs], rows, gsem).wait()

  fire_idx(0, 0)
  wait_idx(0)
  fire_gather(0)

  @pl.when(nk > 1)
  def _():
    fire_idx(1, 1)

  def body(k, _):
    def step(a, b):
      _, idxd_a, rows_a, _, _ = bufs[a]
      wait_gather(a)

      @pl.when(k + 1 < nk)
      def _():
        wait_idx(b)
        fire_gather(b)

      pltpu.sync_copy(rows_a, shared_g.at[idxd_a], add=True)

      @pl.when(k + 2 < nk)
      def _():
        fire_idx(a, k + 2)

    @pl.when(k % 2 == 0)
    def _():
      step(0, 1)

    @pl.when(k % 2 == 1)
    def _():
      step(1, 0)
    return 0
  lax.fori_loop(0, nk, body, 0, unroll=False)
  plsc.subcore_barrier()

  for z in range(NZ):
    r0 = pl.multiple_of(sid * RPT + z * ZCH, ZCH)
    pltpu.sync_copy(shared_g.at[pl.ds(r0, ZCH)], zbuf_v)
    pltpu.sync_copy(zbuf_v, gext_out.at[cid, pl.ds(r0, ZCH)])


def _tc_prep_body(deg_ref, f_ref, h_ref):
  deg = deg_ref[0, :, 0:1] + deg_ref[1, :, 0:1]
  isq = lax.rsqrt(jnp.maximum(deg, 1.0))
  h = jnp.concatenate(
      [f_ref[...] * isq, isq, jnp.zeros((RB, DE - D - 1), jnp.float32)],
      axis=1)
  h_ref[...] = h


def _tc_prep(deg_p, feature):
  return pl.pallas_call(
      _tc_prep_body,
      grid=(NRB,),
      in_specs=[
          pl.BlockSpec((NC, RB, SW), lambda i: (0, i, 0)),
          pl.BlockSpec((RB, D), lambda i: (i, 0)),
      ],
      out_specs=pl.BlockSpec((RB, DE), lambda i: (i, 0)),
      out_shape=jax.ShapeDtypeStruct((N, DE), jnp.float32),
  )(deg_p, feature)


def _tc_final_body(f_ref, h_ref, g_ref, wg_ref, we_ref, o_ref):
  gext = g_ref[0] + g_ref[1]
  hf = h_ref[:, 0:D]
  s1 = gext[:, D:D + 1]
  g = gext[:, 0:D]
  f = f_ref[...]
  a = f + hf * s1
  b = hf * g
  dn = (((1,), (1,)), ((), ()))
  r = lax.dot_general(a, wg_ref[...], dn,
                      preferred_element_type=jnp.float32,
                      precision=lax.Precision.HIGHEST)
  r = r + lax.dot_general(b, we_ref[...], dn,
                          preferred_element_type=jnp.float32,
                          precision=lax.Precision.HIGHEST)
  r = jnp.where(r >= 0, r, 0.2 * r)
  nrm = jnp.sqrt(jnp.sum(r * r, axis=1, keepdims=True))
  o_ref[...] = r / jnp.maximum(nrm, 1e-12)


def _tc_final(feature, hext, g_p, W_gcn, W_enh):
  return pl.pallas_call(
      _tc_final_body,
      grid=(NRB,),
      in_specs=[
          pl.BlockSpec((RB, D), lambda i: (i, 0)),
          pl.BlockSpec((RB, DE), lambda i: (i, 0)),
          pl.BlockSpec((NC, RB, DE), lambda i: (0, i, 0)),
          pl.BlockSpec((D, D), lambda i: (0, 0)),
          pl.BlockSpec((D, D), lambda i: (0, 0)),
      ],
      out_specs=pl.BlockSpec((RB, D), lambda i: (i, 0)),
      out_shape=jax.ShapeDtypeStruct((N, D), jnp.float32),
  )(feature, hext, g_p, W_gcn, W_enh)


def kernel(feature, edge_index, W_gcn, W_enh):
  ei = edge_index.astype(jnp.int32)
  deg_p = _sc_degree(ei)
  hext = _tc_prep(deg_p, feature)
  g_p = _sc_edges(ei, hext)
  return _tc_final(feature, hext, g_p, W_gcn, W_enh)
